# baseline (speedup 1.0000x reference)
import functools

import jax
import jax.numpy as jnp
from jax import lax
from jax.experimental import pallas as pl
from jax.experimental.pallas import tpu as pltpu
from jax.experimental.pallas import tpu_sc as plsc

COMMITMENT_COST = 0.25

B, L, D = 16, 1024, 128
NB, K, DC = 4, 8192, 32
N = B * L
TN = 1024
DCA = DC + 1
NW = 32
CH = (N * NB) // NW
SUB = 4
CHS = CH // SUB


def _argmin_body(zf_ref, ncbt_ref, cnh_ref, idx_ref):
    x = zf_ref[...]
    cols = []
    for i in range(NB):
        xi = x[:, i * DC:(i + 1) * DC]
        ct = ncbt_ref[i * DC:(i + 1) * DC, :]
        s = jax.lax.dot_general(
            xi, ct, (((1,), (0,)), ((), ())),
            preferred_element_type=jnp.float32)
        h = s + cnh_ref[8 * i:8 * i + 1, :]
        cols.append(jnp.argmin(h, axis=1).astype(jnp.int32).reshape(TN, 1))
    idx_ref[...] = jnp.concatenate(cols, axis=1)


def _argmin_call(zf, ncbt, cnh8):
    return pl.pallas_call(
        _argmin_body,
        grid=(N // TN,),
        in_specs=[
            pl.BlockSpec((TN, D), lambda n: (n, 0)),
            pl.BlockSpec((D, K), lambda n: (0, 0)),
            pl.BlockSpec((8 * NB, K), lambda n: (0, 0)),
        ],
        out_specs=pl.BlockSpec((TN, NB), lambda n: (n, 0)),
        out_shape=jax.ShapeDtypeStruct((N, NB), jnp.int32),
    )(zf, ncbt, cnh8)


def _gather_hist_body(idx_hbm, table_hbm, zf_hbm, zq_hbm, hist_hbm, loss_hbm,
                      idx_v, gidx_v, rows_v, z_v, hist_v, acc_v, sem):
    wid = lax.axis_index("s") * 2 + lax.axis_index("c")
    base = wid * CH
    pltpu.sync_copy(idx_hbm.at[pl.ds(base, CH)], idx_v)

    zeros16 = jnp.zeros((16,), jnp.int32)

    def zbody(j, c):
        for u in range(8):
            hist_v[pl.ds(j * 128 + u * 16, 16)] = zeros16
        return c

    lax.fori_loop(0, (NB * K) // 128, zbody, 0)

    offs = (lax.iota(jnp.int32, 16) % NB) * K

    def hbody(j, c):
        for u in range(4):
            v = idx_v[pl.ds(j * 64 + u * 16, 16)] + offs
            gidx_v[pl.ds(j * 64 + u * 16, 16)] = v
            cnt, last = plsc.scan_count(v)
            plsc.addupdate_scatter(hist_v, [v], cnt, mask=last)
        return c

    lax.fori_loop(0, CH // 64, hbody, 0)
    pltpu.sync_copy(hist_v, hist_hbm.at[pl.ds(wid * (NB * K), NB * K)])

    acc = jnp.zeros((16,), jnp.float32)
    for sc in range(SUB):
        lo = sc * CHS
        pltpu.async_copy(table_hbm.at[gidx_v.at[pl.ds(lo, CHS)]], rows_v,
                         sem).wait()
        pltpu.sync_copy(rows_v, zq_hbm.at[pl.ds(base + lo, CHS)])
        pltpu.sync_copy(zf_hbm.at[pl.ds(base + lo, CHS)], z_v)

        def lbody(j, a):
            for u in range(4):
                d0 = z_v[j * 4 + u, pl.ds(0, 16)] - rows_v[j * 4 + u, pl.ds(0, 16)]
                d1 = z_v[j * 4 + u, pl.ds(16, 16)] - rows_v[j * 4 + u, pl.ds(16, 16)]
                a = a + (d0 * d0 + d1 * d1)
            return a

        acc = lax.fori_loop(0, CHS // 4, lbody, acc)
    acc_v[...] = acc
    pltpu.sync_copy(acc_v, loss_hbm.at[pl.ds(wid * 16, 16)])


@functools.cache
def _gather_hist_call():
    return pl.kernel(
        _gather_hist_body,
        out_type=(
            jax.ShapeDtypeStruct((N * NB, DC), jnp.float32),
            jax.ShapeDtypeStruct((NW * NB * K,), jnp.int32),
            jax.ShapeDtypeStruct((NW * 16,), jnp.float32),
        ),
        mesh=plsc.VectorSubcoreMesh(core_axis_name="c", subcore_axis_name="s"),
        compiler_params=pltpu.CompilerParams(
            needs_layout_passes=False, use_tc_tiling_on_sc=False),
        scratch_types=[
            pltpu.VMEM((CH,), jnp.int32),
            pltpu.VMEM((CH,), jnp.int32),
            pltpu.VMEM((CHS, DC), jnp.float32),
            pltpu.VMEM((CHS, DC), jnp.float32),
            pltpu.VMEM((NB * K,), jnp.int32),
            pltpu.VMEM((16,), jnp.float32),
            pltpu.SemaphoreType.DMA,
        ],
    )


def _finalize_body(hist_ref, losspart_ref, loss_ref, ent_ref):
    h = hist_ref[...]
    counts = jnp.sum(h, axis=0, keepdims=True).astype(jnp.float32)
    p = counts / jnp.float32(N)
    ent = -jnp.sum(p * jnp.log(p + 1e-10))
    ent_ref[...] = (ent / jnp.float32(NB)).reshape(1, 1)
    s = jnp.sum(losspart_ref[...])
    loss_ref[...] = ((COMMITMENT_COST * s / jnp.float32(N * DC))
                     / jnp.float32(NB)).reshape(1, 1)


def _finalize_call(hist, loss_part):
    return pl.pallas_call(
        _finalize_body,
        out_shape=[
            jax.ShapeDtypeStruct((1, 1), jnp.float32),
            jax.ShapeDtypeStruct((1, 1), jnp.float32),
        ],
    )(hist, loss_part)


def kernel(z, codebooks):
    zf = z.reshape(N, D)
    ncbt = (-codebooks.transpose(0, 2, 1)).reshape(NB * DC, K)
    cnh = 0.5 * jnp.sum(codebooks * codebooks, axis=2)
    cnh8 = jnp.pad(cnh[:, None, :], ((0, 0), (0, 7), (0, 0))).reshape(8 * NB, K)
    idx = _argmin_call(zf, ncbt, cnh8)
    table = codebooks.reshape(NB * K, DC)
    zq_flat, hist_flat, loss_part = _gather_hist_call()(
        idx.reshape(N * NB), table, zf.reshape(N * NB, DC))
    loss_out, ent_out = _finalize_call(
        hist_flat.reshape(NW, NB * K), loss_part.reshape(NW, 16))
    z_q = zq_flat.reshape(B, L, D)
    indices = idx.reshape(B, L, NB)
    lo = loss_out[0, 0]
    en = ent_out[0, 0]
    return (z_q, indices, lo, lo, en)


# speedup vs baseline: 1.0259x; 1.0259x over previous
import functools

import jax
import jax.numpy as jnp
from jax import lax
from jax.experimental import pallas as pl
from jax.experimental.pallas import tpu as pltpu
from jax.experimental.pallas import tpu_sc as plsc

COMMITMENT_COST = 0.25

B, L, D = 16, 1024, 128
NB, K, DC = 4, 8192, 32
N = B * L
TN = 512
DCA = DC + 1
NW = 32
CH = (N * NB) // NW
SUB = 4
CHS = CH // SUB


def _argmin_body(zf_ref, ncbt_ref, cnh_ref, idx_ref):
    x = zf_ref[...]
    cols = []
    for i in range(NB):
        xi = x[:, i * DC:(i + 1) * DC]
        ct = ncbt_ref[i * DC:(i + 1) * DC, :]
        s = jax.lax.dot_general(
            xi, ct, (((1,), (0,)), ((), ())),
            preferred_element_type=jnp.float32)
        h = s + cnh_ref[8 * i:8 * i + 1, :]
        cols.append(jnp.argmin(h, axis=1).astype(jnp.int32).reshape(TN, 1))
    idx_ref[...] = jnp.concatenate(cols, axis=1)


def _argmin_call(zf, ncbt, cnh8):
    return pl.pallas_call(
        _argmin_body,
        grid=(N // TN,),
        in_specs=[
            pl.BlockSpec((TN, D), lambda n: (n, 0)),
            pl.BlockSpec((D, K), lambda n: (0, 0)),
            pl.BlockSpec((8 * NB, K), lambda n: (0, 0)),
        ],
        out_specs=pl.BlockSpec((TN, NB), lambda n: (n, 0)),
        out_shape=jax.ShapeDtypeStruct((N, NB), jnp.int32),
    )(zf, ncbt, cnh8)


def _gather_hist_body(idx_hbm, table_hbm, zf_hbm, zq_hbm, hist_hbm, loss_hbm,
                      idx_v, gidx_v, rows0_v, rows1_v, z0_v, z1_v, hist_v,
                      acc_v, gsem0, gsem1, zsem0, zsem1):
    wid = lax.axis_index("s") * 2 + lax.axis_index("c")
    base = wid * CH
    pltpu.sync_copy(idx_hbm.at[pl.ds(base, CH)], idx_v)

    offs = (lax.iota(jnp.int32, 16) % NB) * K

    def gbody(j, c):
        for u in range(4):
            sl = pl.ds(j * 64 + u * 16, 16)
            gidx_v[sl] = idx_v[sl] + offs
        return c

    lax.fori_loop(0, CH // 64, gbody, 0)

    rows = (rows0_v, rows1_v)
    zs = (z0_v, z1_v)
    gsems = (gsem0, gsem1)
    zsems = (zsem0, zsem1)

    def fire(sc):
        lo = sc * CHS
        g = pltpu.async_copy(table_hbm.at[gidx_v.at[pl.ds(lo, CHS)]],
                             rows[sc % 2], gsems[sc % 2])
        zc = pltpu.async_copy(zf_hbm.at[pl.ds(base + lo, CHS)],
                              zs[sc % 2], zsems[sc % 2])
        return g, zc

    pending = fire(0)

    zeros16 = jnp.zeros((16,), jnp.int32)

    def zbody(j, c):
        for u in range(8):
            hist_v[pl.ds(j * 128 + u * 16, 16)] = zeros16
        return c

    lax.fori_loop(0, (NB * K) // 128, zbody, 0)

    def hbody(j, c):
        for u in range(4):
            v = gidx_v[pl.ds(j * 64 + u * 16, 16)]
            cnt, last = plsc.scan_count(v)
            plsc.addupdate_scatter(hist_v, [v], cnt, mask=last)
        return c

    lax.fori_loop(0, CH // 64, hbody, 0)
    pltpu.sync_copy(hist_v, hist_hbm.at[pl.ds(wid * (NB * K), NB * K)])

    acc = jnp.zeros((16,), jnp.float32)
    for sc in range(SUB):
        g, zc = pending
        g.wait()
        zc.wait()
        if sc + 1 < SUB:
            pending = fire(sc + 1)
        rv = rows[sc % 2]
        zv = zs[sc % 2]
        pltpu.sync_copy(rv, zq_hbm.at[pl.ds(base + sc * CHS, CHS)])

        def lbody(j, a):
            for u in range(4):
                d0 = zv[j * 4 + u, pl.ds(0, 16)] - rv[j * 4 + u, pl.ds(0, 16)]
                d1 = zv[j * 4 + u, pl.ds(16, 16)] - rv[j * 4 + u, pl.ds(16, 16)]
                a = a + (d0 * d0 + d1 * d1)
            return a

        acc = lax.fori_loop(0, CHS // 4, lbody, acc)
    acc_v[...] = acc
    pltpu.sync_copy(acc_v, loss_hbm.at[pl.ds(wid * 16, 16)])


@functools.cache
def _gather_hist_call():
    return pl.kernel(
        _gather_hist_body,
        out_type=(
            jax.ShapeDtypeStruct((N * NB, DC), jnp.float32),
            jax.ShapeDtypeStruct((NW * NB * K,), jnp.int32),
            jax.ShapeDtypeStruct((NW * 16,), jnp.float32),
        ),
        mesh=plsc.VectorSubcoreMesh(core_axis_name="c", subcore_axis_name="s"),
        compiler_params=pltpu.CompilerParams(
            needs_layout_passes=False, use_tc_tiling_on_sc=False),
        scratch_types=[
            pltpu.VMEM((CH,), jnp.int32),
            pltpu.VMEM((CH,), jnp.int32),
            pltpu.VMEM((CHS, DC), jnp.float32),
            pltpu.VMEM((CHS, DC), jnp.float32),
            pltpu.VMEM((CHS, DC), jnp.float32),
            pltpu.VMEM((CHS, DC), jnp.float32),
            pltpu.VMEM((NB * K,), jnp.int32),
            pltpu.VMEM((16,), jnp.float32),
            pltpu.SemaphoreType.DMA,
            pltpu.SemaphoreType.DMA,
            pltpu.SemaphoreType.DMA,
            pltpu.SemaphoreType.DMA,
        ],
    )


def _finalize_body(hist_ref, losspart_ref, loss_ref, ent_ref):
    h = hist_ref[...]
    counts = jnp.sum(h, axis=0, keepdims=True).astype(jnp.float32)
    p = counts / jnp.float32(N)
    ent = -jnp.sum(p * jnp.log(p + 1e-10))
    ent_ref[...] = (ent / jnp.float32(NB)).reshape(1, 1)
    s = jnp.sum(losspart_ref[...])
    loss_ref[...] = ((COMMITMENT_COST * s / jnp.float32(N * DC))
                     / jnp.float32(NB)).reshape(1, 1)


def _finalize_call(hist, loss_part):
    return pl.pallas_call(
        _finalize_body,
        out_shape=[
            jax.ShapeDtypeStruct((1, 1), jnp.float32),
            jax.ShapeDtypeStruct((1, 1), jnp.float32),
        ],
    )(hist, loss_part)


def kernel(z, codebooks):
    zf = z.reshape(N, D)
    ncbt = (-codebooks.transpose(0, 2, 1)).reshape(NB * DC, K)
    cnh = 0.5 * jnp.sum(codebooks * codebooks, axis=2)
    cnh8 = jnp.pad(cnh[:, None, :], ((0, 0), (0, 7), (0, 0))).reshape(8 * NB, K)
    idx = _argmin_call(zf, ncbt, cnh8)
    table = codebooks.reshape(NB * K, DC)
    zq_flat, hist_flat, loss_part = _gather_hist_call()(
        idx.reshape(N * NB), table, zf.reshape(N * NB, DC))
    loss_out, ent_out = _finalize_call(
        hist_flat.reshape(NW, NB * K), loss_part.reshape(NW, 16))
    z_q = zq_flat.reshape(B, L, D)
    indices = idx.reshape(B, L, NB)
    lo = loss_out[0, 0]
    en = ent_out[0, 0]
    return (z_q, indices, lo, lo, en)
